# Initial kernel scaffold; baseline (speedup 1.0000x reference)
import jax
import jax.numpy as jnp
from jax.experimental import pallas as pl


def kernel(points, params):
    raise NotImplementedError("write your pallas kernel here")


# speedup vs baseline: 2.9957x; 2.9957x over previous
import functools
import jax
import jax.numpy as jnp
from jax.experimental import pallas as pl

_B = 2
_N0 = 8192
_BIG = 1e30

_CFGS = [
    {"npoint": 2048, "radii": (0.2, 0.8), "nsamples": (16, 32)},
    {"npoint": 512, "radii": (0.8, 1.6), "nsamples": (16, 32)},
]


def _fps_kernel(x_ref, y_ref, z_ref, oi_ref, ox_ref, oy_ref, oz_ref, *, n, m):
    x = x_ref[0]
    y = y_ref[0]
    z = z_ref[0]
    iota_n = jax.lax.broadcasted_iota(jnp.int32, (1, n), 1)
    iota_m = jax.lax.broadcasted_iota(jnp.int32, (1, m), 1)

    def step(t, carry):
        d, far, oi, ox, oy, oz = carry
        msk = iota_n == far
        cx = jnp.sum(jnp.where(msk, x, 0.0))
        cy = jnp.sum(jnp.where(msk, y, 0.0))
        cz = jnp.sum(jnp.where(msk, z, 0.0))
        sel = iota_m == t
        oi = jnp.where(sel, far, oi)
        ox = jnp.where(sel, cx, ox)
        oy = jnp.where(sel, cy, oy)
        oz = jnp.where(sel, cz, oz)
        dx = x - cx
        dy = y - cy
        dz = z - cz
        dist = (dx * dx + dy * dy) + dz * dz
        d = jnp.minimum(d, dist)
        far = jnp.argmax(d[0], axis=0).astype(jnp.int32)
        return d, far, oi, ox, oy, oz

    d0 = jnp.full((1, n), 1e10, dtype=jnp.float32)
    init = (d0, jnp.int32(0), jnp.zeros((1, m), jnp.int32),
            jnp.zeros((1, m), jnp.float32), jnp.zeros((1, m), jnp.float32),
            jnp.zeros((1, m), jnp.float32))
    _, _, oi, ox, oy, oz = jax.lax.fori_loop(0, m, step, init)
    oi_ref[0] = oi
    ox_ref[0] = ox
    oy_ref[0] = oy
    oz_ref[0] = oz


def _fps_call(x, y, z, m):
    b, n = x.shape
    x3 = x.reshape(b, 1, n)
    y3 = y.reshape(b, 1, n)
    z3 = z.reshape(b, 1, n)
    in_spec = pl.BlockSpec((1, 1, n), lambda i: (i, 0, 0))
    out_spec = pl.BlockSpec((1, 1, m), lambda i: (i, 0, 0))
    oi, ox, oy, oz = pl.pallas_call(
        functools.partial(_fps_kernel, n=n, m=m),
        grid=(b,),
        in_specs=[in_spec, in_spec, in_spec],
        out_specs=[out_spec, out_spec, out_spec, out_spec],
        out_shape=[
            jax.ShapeDtypeStruct((b, 1, m), jnp.int32),
            jax.ShapeDtypeStruct((b, 1, m), jnp.float32),
            jax.ShapeDtypeStruct((b, 1, m), jnp.float32),
            jax.ShapeDtypeStruct((b, 1, m), jnp.float32),
        ],
    )(x3, y3, z3)
    return (oi.reshape(b, m), ox.reshape(b, m), oy.reshape(b, m),
            oz.reshape(b, m))


def _knn_kernel(cx_ref, cy_ref, cz_ref, x_ref, y_ref, z_ref,
                oi_ref, *, n, bm, k, ks, r2, r2s):
    cx = cx_ref[0]
    cy = cy_ref[0]
    cz = cz_ref[0]
    x = x_ref[0]
    y = y_ref[0]
    z = z_ref[0]
    aa = (cx * cx + cy * cy) + cz * cz
    bb = (x * x + y * y) + z * z
    cen = jnp.concatenate([cx, cy, cz], axis=1).astype(jnp.bfloat16)
    pts = jnp.concatenate([x, y, z], axis=0).astype(jnp.bfloat16)
    ab = jnp.dot(cen, pts, preferred_element_type=jnp.float32)
    d2 = (aa - 2.0 * ab) + bb
    lane_n = jax.lax.broadcasted_iota(jnp.int32, (bm, n), 1)
    lane_k = jax.lax.broadcasted_iota(jnp.int32, (bm, k), 1)

    a0 = jnp.argmin(d2, axis=1).astype(jnp.int32)
    m0 = jnp.min(d2, axis=1)
    oi = jnp.broadcast_to(a0[:, None], (bm, k))
    od = jnp.where(lane_k == 0, m0[:, None], _BIG)
    d2 = jnp.where((d2 <= r2) & (lane_n != a0[:, None]), d2, _BIG)

    def cond(s):
        return s[0]

    def body(s):
        _, t, d2, oi, od = s
        m = jnp.min(d2, axis=1)
        a = jnp.argmin(d2, axis=1).astype(jnp.int32)
        valid = m < _BIG
        sel = (lane_k == t) & valid[:, None]
        oi = jnp.where(sel, a[:, None], oi)
        od = jnp.where(sel, m[:, None], od)
        d2 = jnp.where((lane_n == a[:, None]) & valid[:, None], _BIG, d2)
        t = t + 1
        go = (t < k) & jnp.any(valid)
        return go, t, d2, oi, od

    _, _, _, oi, od = jax.lax.while_loop(
        cond, body, (jnp.bool_(True), jnp.int32(1), d2, oi, od))
    ism = jnp.where(od[:, :ks] <= r2s, oi[:, :ks], oi[:, :1])
    oi_ref[0] = jnp.concatenate([oi, ism], axis=1)


def _knn_call(cx, cy, cz, x, y, z, k, ks, r, rs, bm):
    b, msz = cx.shape
    n = x.shape[1]
    nblk = msz // bm
    cset = [c.reshape(b * nblk, bm, 1) for c in (cx, cy, cz)]
    pset = [p.reshape(b, 1, n) for p in (x, y, z)]
    c_spec = pl.BlockSpec((1, bm, 1), lambda i, j: (i * nblk + j, 0, 0))
    p_spec = pl.BlockSpec((1, 1, n), lambda i, j: (i, 0, 0))
    o_spec = pl.BlockSpec((1, bm, k + ks), lambda i, j: (i * nblk + j, 0, 0))
    oi = pl.pallas_call(
        functools.partial(_knn_kernel, n=n, bm=bm, k=k, ks=ks,
                          r2=r * r, r2s=rs * rs),
        grid=(b, nblk),
        in_specs=[c_spec, c_spec, c_spec, p_spec, p_spec, p_spec],
        out_specs=o_spec,
        out_shape=jax.ShapeDtypeStruct((b * nblk, bm, k + ks), jnp.int32),
    )(*cset, *pset)
    return oi.reshape(b, msz, k + ks)


def _mlp_kernel(*refs, bm, k, ks, d, dims1, dims2):
    g_ref, c_ref = refs[0], refs[1]
    wrefs = refs[2:-1]
    o_ref = refs[-1]
    g3 = g_ref[...]
    cen = c_ref[...]
    cpad = jnp.concatenate(
        [cen, jnp.zeros((bm, d - 3), jnp.float32)], axis=1)
    g3 = g3 - cpad[:, None, :]

    wi = 0

    def run_mlp(rows, nlayers):
        nonlocal wi
        h = rows
        for _ in range(nlayers):
            w = wrefs[wi][...]
            bvec = wrefs[wi + 1][...]
            wi += 2
            h = jnp.maximum(
                jnp.dot(h, w, preferred_element_type=jnp.float32) + bvec, 0.0)
        return h

    gs = g3[:, k:, :]
    h1 = run_mlp(gs.reshape(bm * ks, d), len(dims1) - 1)
    f1 = jnp.max(h1.reshape(bm, ks, dims1[-1]), axis=1)

    h2 = run_mlp(g3[:, :k, :].reshape(bm * k, d), len(dims2) - 1)
    f2 = jnp.max(h2.reshape(bm, k, dims2[-1]), axis=1)

    f = jnp.concatenate([f1, f2], axis=1)
    wagg = wrefs[wi][...]
    bagg = wrefs[wi + 1][...]
    o_ref[...] = jnp.maximum(
        jnp.dot(f, wagg, preferred_element_type=jnp.float32) + bagg, 0.0)


def _mlp_call(gathered, centers, p, dims1, dims2, ks, bm):
    rtot, ktot, d = gathered.shape
    k = ktot - ks
    nblk = rtot // bm
    warrs = []
    for ws in p["scales"]:
        for lyr in ws:
            warrs.append(lyr["W"])
            warrs.append(lyr["b"].reshape(1, -1))
    warrs.append(p["agg"]["W"])
    warrs.append(p["agg"]["b"].reshape(1, -1))
    cout = p["agg"]["W"].shape[1]
    w_specs = [pl.BlockSpec(w.shape, lambda i: (0,) * w.ndim) for w in warrs]
    out = pl.pallas_call(
        functools.partial(_mlp_kernel, bm=bm, k=k, ks=ks, d=d,
                          dims1=dims1, dims2=dims2),
        grid=(nblk,),
        in_specs=[
            pl.BlockSpec((bm, ktot, d), lambda i: (i, 0, 0)),
            pl.BlockSpec((bm, 3), lambda i: (i, 0)),
        ] + w_specs,
        out_specs=pl.BlockSpec((bm, cout), lambda i: (i, 0)),
        out_shape=jax.ShapeDtypeStruct((rtot, cout), jnp.float32),
    )(gathered, centers, *warrs)
    return out


def _gather_rows(table, idx):
    return jax.vmap(lambda t, i: t[i])(table, idx.reshape(idx.shape[0], -1)
                                       ).reshape(idx.shape + (table.shape[-1],))


def _sa_layer(xyz, feats, cfg, p, dims1, dims2):
    b, n, _ = xyz.shape
    m = cfg["npoint"]
    x, y, z = xyz[..., 0], xyz[..., 1], xyz[..., 2]
    _, ncx, ncy, ncz = _fps_call(x, y, z, m)
    ks, k = cfg["nsamples"]
    oi = _knn_call(ncx, ncy, ncz, x, y, z, k, ks,
                   cfg["radii"][1], cfg["radii"][0], bm=256)
    table = jnp.concatenate([xyz, feats], axis=-1)
    gathered = _gather_rows(table, oi)
    d = table.shape[-1]
    centers = jnp.stack([ncx, ncy, ncz], axis=-1)
    out = _mlp_call(gathered.reshape(b * m, k + ks, d),
                    centers.reshape(b * m, 3),
                    p, dims1, dims2, ks=ks, bm=256)
    return centers, out.reshape(b, m, -1)


_DIMS = [
    ([4, 16, 16, 32], [4, 32, 32, 64]),
    ([67, 64, 64, 128], [67, 64, 96, 128]),
]


def kernel(points, params):
    xyz = points[:, 1:4].reshape(_B, _N0, 3)
    feats = points[:, 4:].reshape(_B, _N0, -1)
    for cfg, p, (dims1, dims2) in zip(_CFGS, params, _DIMS):
        xyz, feats = _sa_layer(xyz, feats, cfg, p, dims1, dims2)
    return feats


# speedup vs baseline: 6.0500x; 2.0196x over previous
import functools
import jax
import jax.numpy as jnp
from jax.experimental import pallas as pl
from jax.experimental.pallas import tpu as pltpu
from jax.experimental.pallas import tpu_sc as plsc

_B = 2
_N0 = 8192
_BIG = 1e30

_CFGS = [
    {"npoint": 2048, "radii": (0.2, 0.8), "nsamples": (16, 32)},
    {"npoint": 512, "radii": (0.8, 1.6), "nsamples": (16, 32)},
]


def _fps_kernel(x_ref, y_ref, z_ref, oi_ref, ox_ref, oy_ref, oz_ref, *, n, m):
    x = x_ref[0]
    y = y_ref[0]
    z = z_ref[0]
    iota_n = jax.lax.broadcasted_iota(jnp.int32, (1, n), 1)
    iota_m = jax.lax.broadcasted_iota(jnp.int32, (1, m), 1)

    def step(t, carry):
        d, far, oi, ox, oy, oz = carry
        msk = iota_n == far
        cx = jnp.sum(jnp.where(msk, x, 0.0))
        cy = jnp.sum(jnp.where(msk, y, 0.0))
        cz = jnp.sum(jnp.where(msk, z, 0.0))
        sel = iota_m == t
        oi = jnp.where(sel, far, oi)
        ox = jnp.where(sel, cx, ox)
        oy = jnp.where(sel, cy, oy)
        oz = jnp.where(sel, cz, oz)
        dx = x - cx
        dy = y - cy
        dz = z - cz
        dist = (dx * dx + dy * dy) + dz * dz
        d = jnp.minimum(d, dist)
        far = jnp.argmax(d[0], axis=0).astype(jnp.int32)
        return d, far, oi, ox, oy, oz

    d0 = jnp.full((1, n), 1e10, dtype=jnp.float32)
    init = (d0, jnp.int32(0), jnp.zeros((1, m), jnp.int32),
            jnp.zeros((1, m), jnp.float32), jnp.zeros((1, m), jnp.float32),
            jnp.zeros((1, m), jnp.float32))
    _, _, oi, ox, oy, oz = jax.lax.fori_loop(0, m, step, init)
    oi_ref[0] = oi
    ox_ref[0] = ox
    oy_ref[0] = oy
    oz_ref[0] = oz


def _fps_call(x, y, z, m):
    b, n = x.shape
    x3 = x.reshape(b, 1, n)
    y3 = y.reshape(b, 1, n)
    z3 = z.reshape(b, 1, n)
    in_spec = pl.BlockSpec((1, 1, n), lambda i: (i, 0, 0))
    out_spec = pl.BlockSpec((1, 1, m), lambda i: (i, 0, 0))
    oi, ox, oy, oz = pl.pallas_call(
        functools.partial(_fps_kernel, n=n, m=m),
        grid=(b,),
        in_specs=[in_spec, in_spec, in_spec],
        out_specs=[out_spec, out_spec, out_spec, out_spec],
        out_shape=[
            jax.ShapeDtypeStruct((b, 1, m), jnp.int32),
            jax.ShapeDtypeStruct((b, 1, m), jnp.float32),
            jax.ShapeDtypeStruct((b, 1, m), jnp.float32),
            jax.ShapeDtypeStruct((b, 1, m), jnp.float32),
        ],
    )(x3, y3, z3)
    return (oi.reshape(b, m), ox.reshape(b, m), oy.reshape(b, m),
            oz.reshape(b, m))


def _knn_kernel(cx_ref, cy_ref, cz_ref, x_ref, y_ref, z_ref,
                oi_ref, *, n, bm, k, ks, r2, r2s):
    bid = pl.program_id(0)
    cx = cx_ref[0]
    cy = cy_ref[0]
    cz = cz_ref[0]
    x = x_ref[0]
    y = y_ref[0]
    z = z_ref[0]
    aa = (cx * cx + cy * cy) + cz * cz
    bb = (x * x + y * y) + z * z
    cen = jnp.concatenate([cx, cy, cz], axis=1).astype(jnp.bfloat16)
    pts = jnp.concatenate([x, y, z], axis=0).astype(jnp.bfloat16)
    ab = jnp.dot(cen, pts, preferred_element_type=jnp.float32)
    d2 = (aa - 2.0 * ab) + bb
    lane_n = jax.lax.broadcasted_iota(jnp.int32, (bm, n), 1)
    lane_k = jax.lax.broadcasted_iota(jnp.int32, (bm, k), 1)

    a0 = jnp.argmin(d2, axis=1).astype(jnp.int32)
    m0 = jnp.min(d2, axis=1)
    oi = jnp.broadcast_to(a0[:, None], (bm, k))
    od = jnp.where(lane_k == 0, m0[:, None], _BIG)
    d2 = jnp.where((d2 <= r2) & (lane_n != a0[:, None]), d2, _BIG)

    def cond(s):
        return s[0]

    def body(s):
        _, t, d2, oi, od = s
        m = jnp.min(d2, axis=1)
        a = jnp.argmin(d2, axis=1).astype(jnp.int32)
        valid = m < _BIG
        sel = (lane_k == t) & valid[:, None]
        oi = jnp.where(sel, a[:, None], oi)
        od = jnp.where(sel, m[:, None], od)
        d2 = jnp.where((lane_n == a[:, None]) & valid[:, None], _BIG, d2)
        t = t + 1
        go = (t < k) & jnp.any(valid)
        return go, t, d2, oi, od

    _, _, _, oi, od = jax.lax.while_loop(
        cond, body, (jnp.bool_(True), jnp.int32(1), d2, oi, od))
    ism = jnp.where(od[:, :ks] <= r2s, oi[:, :ks], oi[:, :1])
    oi_ref[0] = jnp.concatenate([oi, ism], axis=1) + bid * n


def _knn_call(cx, cy, cz, x, y, z, k, ks, r, rs, bm):
    b, msz = cx.shape
    n = x.shape[1]
    nblk = msz // bm
    cset = [c.reshape(b * nblk, bm, 1) for c in (cx, cy, cz)]
    pset = [p.reshape(b, 1, n) for p in (x, y, z)]
    c_spec = pl.BlockSpec((1, bm, 1), lambda i, j: (i * nblk + j, 0, 0))
    p_spec = pl.BlockSpec((1, 1, n), lambda i, j: (i, 0, 0))
    o_spec = pl.BlockSpec((1, bm, k + ks), lambda i, j: (i * nblk + j, 0, 0))
    oi = pl.pallas_call(
        functools.partial(_knn_kernel, n=n, bm=bm, k=k, ks=ks,
                          r2=r * r, r2s=rs * rs),
        grid=(b, nblk),
        in_specs=[c_spec, c_spec, c_spec, p_spec, p_spec, p_spec],
        out_specs=o_spec,
        out_shape=jax.ShapeDtypeStruct((b * nblk, bm, k + ks), jnp.int32),
    )(*cset, *pset)
    return oi.reshape(b, msz, k + ks)


def _mlp_kernel(*refs, bm, k, ks, d, dims1, dims2):
    g_ref, c_ref = refs[0], refs[1]
    wrefs = refs[2:-1]
    o_ref = refs[-1]
    g3 = g_ref[...]
    cen = c_ref[...]
    cpad = jnp.concatenate(
        [cen, jnp.zeros((bm, d - 3), jnp.float32)], axis=1)
    g3 = g3 - cpad[:, None, :]

    wi = 0

    def run_mlp(rows, nlayers):
        nonlocal wi
        h = rows
        for _ in range(nlayers):
            w = wrefs[wi][...]
            bvec = wrefs[wi + 1][...]
            wi += 2
            h = jnp.maximum(
                jnp.dot(h, w, preferred_element_type=jnp.float32) + bvec, 0.0)
        return h

    gs = g3[:, k:, :]
    h1 = run_mlp(gs.reshape(bm * ks, d), len(dims1) - 1)
    f1 = jnp.max(h1.reshape(bm, ks, dims1[-1]), axis=1)

    h2 = run_mlp(g3[:, :k, :].reshape(bm * k, d), len(dims2) - 1)
    f2 = jnp.max(h2.reshape(bm, k, dims2[-1]), axis=1)

    f = jnp.concatenate([f1, f2], axis=1)
    wagg = wrefs[wi][...]
    bagg = wrefs[wi + 1][...]
    o_ref[...] = jnp.maximum(
        jnp.dot(f, wagg, preferred_element_type=jnp.float32) + bagg, 0.0)


def _mlp_call(gathered, centers, p, dims1, dims2, ks, bm):
    rtot, ktot, d = gathered.shape
    k = ktot - ks
    nblk = rtot // bm
    warrs = []
    for ws in p["scales"]:
        for li, lyr in enumerate(ws):
            w = lyr["W"]
            if li == 0 and w.shape[0] < d:
                w = jnp.concatenate(
                    [w, jnp.zeros((d - w.shape[0], w.shape[1]), w.dtype)], 0)
            warrs.append(w)
            warrs.append(lyr["b"].reshape(1, -1))
    warrs.append(p["agg"]["W"])
    warrs.append(p["agg"]["b"].reshape(1, -1))
    cout = p["agg"]["W"].shape[1]
    w_specs = [pl.BlockSpec(w.shape, lambda i: (0,) * w.ndim) for w in warrs]
    out = pl.pallas_call(
        functools.partial(_mlp_kernel, bm=bm, k=k, ks=ks, d=d,
                          dims1=dims1, dims2=dims2),
        grid=(nblk,),
        in_specs=[
            pl.BlockSpec((bm, ktot, d), lambda i: (i, 0, 0)),
            pl.BlockSpec((bm, 3), lambda i: (i, 0)),
        ] + w_specs,
        out_specs=pl.BlockSpec((bm, cout), lambda i: (i, 0)),
        out_shape=jax.ShapeDtypeStruct((rtot, cout), jnp.float32),
    )(gathered, centers, *warrs)
    return out


def _sc_gather(table, idx):
    rtot = idx.shape[0]
    d = table.shape[1]
    info = plsc.get_sparse_core_info()
    nw = info.num_cores * info.num_subcores
    rows_w = rtot // nw
    nch = rows_w // 128
    idx2 = idx.reshape(nw * nch, 128)
    mesh = plsc.VectorSubcoreMesh(core_axis_name="c", subcore_axis_name="s")

    @functools.partial(
        pl.kernel, mesh=mesh,
        out_type=jax.ShapeDtypeStruct((rtot, d), jnp.float32),
        compiler_params=pltpu.CompilerParams(use_tc_tiling_on_sc=False),
        scratch_types=[
            pltpu.VMEM((nch, 128), jnp.int32),
            pltpu.VMEM((rows_w, d), jnp.float32),
            pltpu.SemaphoreType.DMA,
        ],
    )
    def k(table_hbm, idx_hbm, out_hbm, idx_v, rows_v, sem):
        wid = jax.lax.axis_index("s") * info.num_cores + jax.lax.axis_index("c")
        pltpu.sync_copy(idx_hbm.at[pl.ds(wid * nch, nch)], idx_v)
        cps = []
        for j in range(nch):
            cps.append(pltpu.async_copy(
                table_hbm.at[idx_v.at[j]],
                rows_v.at[pl.ds(j * 128, 128)], sem))
        for cp in cps:
            cp.wait()
        pltpu.sync_copy(rows_v, out_hbm.at[pl.ds(wid * rows_w, rows_w)])

    return k(table, idx2)


def _sa_layer(xyz, feats, cfg, p, dims1, dims2):
    b, n, _ = xyz.shape
    m = cfg["npoint"]
    x, y, z = xyz[..., 0], xyz[..., 1], xyz[..., 2]
    _, ncx, ncy, ncz = _fps_call(x, y, z, m)
    ks, k = cfg["nsamples"]
    oi = _knn_call(ncx, ncy, ncz, x, y, z, k, ks,
                   cfg["radii"][1], cfg["radii"][0], bm=256)
    dtrue = 3 + feats.shape[-1]
    dpad = (dtrue + 15) // 16 * 16
    table = jnp.concatenate(
        [xyz, feats, jnp.zeros((b, n, dpad - dtrue), jnp.float32)],
        axis=-1).reshape(b * n, dpad)
    gathered = _sc_gather(table, oi.reshape(-1))
    centers = jnp.stack([ncx, ncy, ncz], axis=-1)
    out = _mlp_call(gathered.reshape(b * m, k + ks, dpad),
                    centers.reshape(b * m, 3),
                    p, dims1, dims2, ks=ks, bm=256)
    return centers, out.reshape(b, m, -1)


_DIMS = [
    ([4, 16, 16, 32], [4, 32, 32, 64]),
    ([67, 64, 64, 128], [67, 64, 96, 128]),
]


def kernel(points, params):
    xyz = points[:, 1:4].reshape(_B, _N0, 3)
    feats = points[:, 4:].reshape(_B, _N0, -1)
    for cfg, p, (dims1, dims2) in zip(_CFGS, params, _DIMS):
        xyz, feats = _sa_layer(xyz, feats, cfg, p, dims1, dims2)
    return feats
